# Initial kernel scaffold; baseline (speedup 1.0000x reference)
import jax
import jax.numpy as jnp
from jax.experimental import pallas as pl


def kernel(params, x, edge_index, edge_attr, batch_idx):
    raise NotImplementedError("write your pallas kernel here")


# speedup vs baseline: 5.5019x; 5.5019x over previous
import functools

import jax
import jax.numpy as jnp
import numpy as np
from jax import lax
from jax.experimental import pallas as pl
from jax.experimental.pallas import tpu as pltpu
from jax.experimental.pallas import tpu_sc as plsc

N = 10000
E = 160000
G = 128
H = 256
HH = 128
IN_DIM = 72
ALPHA = 0.01
ATOM_OFF = (0, 44, 51, 56, 62, 64, 66)
EDGE_OFF = (0, 4, 6, 8)

_SCALE = np.float32(min(1.0, 1.0 / np.float32(1.0 + 1e-7)))
VNODE = np.float32(_SCALE / np.sqrt(np.float32(7) * _SCALE * _SCALE, dtype=np.float32))
VEDGE = np.float32(_SCALE / np.sqrt(np.float32(4) * _SCALE * _SCALE, dtype=np.float32))

NC, NS = 2, 16
EPT = E // NS
RB = 624
C = 80
NCH = EPT // C
VR_MAIN = 313
VR_LAST = 297

_mesh = plsc.VectorSubcoreMesh(
    core_axis_name="c", subcore_axis_name="s", num_cores=NC, num_subcores=NS)


def _pre_body(x_ref, ea_ref, w0_ref, we_ref, be_ref, hw0_ref, ew_ref):
    acc = jnp.zeros((N, H), jnp.float32)
    for i in range(7):
        r0 = w0_ref[ATOM_OFF[i]:ATOM_OFF[i] + 1, :]
        r1 = w0_ref[ATOM_OFF[i] + 1:ATOM_OFF[i] + 2, :]
        xi = x_ref[:, i:i + 1].astype(jnp.float32)
        acc = acc + r0 + xi * (r1 - r0)
    hw0 = VNODE * acc
    hw0_ref[0] = hw0[:, :HH]
    hw0_ref[1] = hw0[:, HH:]
    ew = jnp.full((1, E), be_ref[0, 0], jnp.float32)
    for i in range(4):
        w0 = we_ref[0, EDGE_OFF[i]]
        w1 = we_ref[0, EDGE_OFF[i] + 1]
        ai = ea_ref[i:i + 1, :].astype(jnp.float32)
        ew = ew + VEDGE * (w0 + ai * (w1 - w0))
    ew_ref[...] = ew


_pre_call = pl.pallas_call(
    _pre_body,
    out_shape=(jax.ShapeDtypeStruct((2, N, HH), jnp.float32),
               jax.ShapeDtypeStruct((1, E), jnp.float32)),
    in_specs=[pl.BlockSpec(), pl.BlockSpec(), pl.BlockSpec(),
              pl.BlockSpec(memory_space=pltpu.SMEM),
              pl.BlockSpec(memory_space=pltpu.SMEM)],
)


@functools.partial(
    pl.kernel,
    out_type=[jax.ShapeDtypeStruct((N,), jnp.float32),
              jax.ShapeDtypeStruct((N,), jnp.float32),
              jax.ShapeDtypeStruct((E,), jnp.float32)],
    mesh=_mesh,
    compiler_params=pltpu.CompilerParams(needs_layout_passes=False),
    scratch_types=[pltpu.VMEM((N,), jnp.float32),
                   pltpu.VMEM((N,), jnp.float32),
                   pltpu.VMEM((N,), jnp.float32),
                   pltpu.VMEM((EPT,), jnp.int32),
                   pltpu.VMEM((EPT,), jnp.float32),
                   pltpu.VMEM((VR_MAIN * 16,), jnp.float32),
                   pltpu.VMEM_SHARED((NS, N), jnp.float32)],
)
def _sc_prep(src_hbm, dst_hbm, ew_hbm, dinv_hbm, dinv2_hbm, ews_hbm,
             deg_v, dinv_v, tmp_v, idx_v, ewc_v, ews_v, deg_sh):
    cc = lax.axis_index("c")
    ss = lax.axis_index("s")
    wid = cc * NS + ss

    def zero_body(j, _):
        deg_v[pl.ds(j * 16, 16)] = jnp.zeros((16,), jnp.float32)
        return 0
    lax.fori_loop(0, N // 16, zero_body, 0)

    base = ss * EPT
    pltpu.sync_copy(dst_hbm.at[pl.ds(base, EPT)], idx_v)
    pltpu.sync_copy(ew_hbm.at[pl.ds(base, EPT)], ewc_v)

    def acc_body(j, _):
        dv = idx_v[pl.ds(j * 16, 16)]
        wv = ewc_v[pl.ds(j * 16, 16)]
        plsc.addupdate_scatter(deg_v, [dv], wv)
        return 0
    lax.fori_loop(0, EPT // 16, acc_body, 0)
    pltpu.sync_copy(deg_v, deg_sh.at[ss])
    plsc.subcore_barrier()
    for r in range(NS):
        pltpu.sync_copy(deg_sh.at[r], tmp_v)
        if r == 0:
            def sum0_body(j, _):
                deg_v[pl.ds(j * 16, 16)] = tmp_v[pl.ds(j * 16, 16)]
                return 0
        else:
            def sum0_body(j, _):
                sl = pl.ds(j * 16, 16)
                deg_v[sl] = deg_v[sl] + tmp_v[sl]
                return 0
        lax.fori_loop(0, N // 16, sum0_body, 0)

    def rsqrt_body(j, _):
        d = deg_v[pl.ds(j * 16, 16)] + 1.0
        bi = plsc.bitcast(d, jnp.int32)
        bi = 0x5F3759DF - lax.shift_right_logical(bi, 1)
        y = plsc.bitcast(bi, jnp.float32)
        for _ in range(4):
            y = y * (1.5 - 0.5 * d * y * y)
        dinv_v[pl.ds(j * 16, 16)] = y
        deg_v[pl.ds(j * 16, 16)] = 1.0 / d
        return 0
    lax.fori_loop(0, N // 16, rsqrt_body, 0)

    @pl.when(wid == 0)
    def _():
        pltpu.sync_copy(dinv_v, dinv_hbm)
        pltpu.sync_copy(deg_v, dinv2_hbm)

    def ews_chunk(nvr):
        ebase = wid * (VR_MAIN * 16)
        ne = nvr * 16
        pltpu.sync_copy(src_hbm.at[pl.ds(ebase, ne)], idx_v.at[pl.ds(0, ne)])
        pltpu.sync_copy(ew_hbm.at[pl.ds(ebase, ne)], ewc_v.at[pl.ds(0, ne)])

        def g_body(j, _):
            sv = idx_v[pl.ds(j * 16, 16)]
            dv = plsc.load_gather(dinv_v, [sv])
            ews_v[pl.ds(j * 16, 16)] = ewc_v[pl.ds(j * 16, 16)] * dv
            return 0
        lax.fori_loop(0, nvr, g_body, 0)
        pltpu.sync_copy(ews_v.at[pl.ds(0, ne)], ews_hbm.at[pl.ds(ebase, ne)])

    @pl.when(wid < NC * NS - 1)
    def _():
        ews_chunk(VR_MAIN)

    @pl.when(wid == NC * NS - 1)
    def _():
        ews_chunk(VR_LAST)


@functools.partial(
    pl.kernel,
    out_type=jax.ShapeDtypeStruct((2, N, HH), jnp.float32),
    mesh=_mesh,
    compiler_params=pltpu.CompilerParams(needs_layout_passes=False),
    scratch_types=[pltpu.VMEM((C,), jnp.int32),
                   pltpu.VMEM((C,), jnp.int32),
                   pltpu.VMEM((C,), jnp.float32),
                   pltpu.VMEM((C, HH), jnp.float32),
                   pltpu.VMEM((160, HH), jnp.float32),
                   pltpu.VMEM_SHARED((N, HH), jnp.float32),
                   pltpu.SemaphoreType.DMA],
)
def _sc_agg(hw_hbm, src_hbm, dst_hbm, ews_hbm, out_hbm,
            sidx_v, didx_v, ewc_v, rows_v, zb_v, acc_sh, sem):
    cc = lax.axis_index("c")
    ss = lax.axis_index("s")

    def zb_body(j, _):
        for k in range(HH // 16):
            zb_v[j, pl.ds(k * 16, 16)] = jnp.zeros((16,), jnp.float32)
        return 0
    lax.fori_loop(0, 160, zb_body, 0)

    @pl.when(ss < NS - 1)
    def _():
        for q in range(4):
            pltpu.sync_copy(zb_v.at[pl.ds(0, 156)],
                            acc_sh.at[pl.ds(ss * RB + q * 156, 156)])

    @pl.when(ss == NS - 1)
    def _():
        for q in range(4):
            pltpu.sync_copy(zb_v, acc_sh.at[pl.ds(ss * RB + q * 160, 160)])
    plsc.subcore_barrier()

    ebase = ss * EPT

    def chunk_body(ch, _):
        b = ebase + ch * C
        pltpu.sync_copy(src_hbm.at[pl.ds(b, C)], sidx_v)
        pltpu.sync_copy(dst_hbm.at[pl.ds(b, C)], didx_v)
        pltpu.sync_copy(ews_hbm.at[pl.ds(b, C)], ewc_v)
        pltpu.async_copy(hw_hbm.at[cc].at[sidx_v], rows_v, sem).wait()

        def mul_body(j, _):
            bb = plsc.load_gather(ewc_v, [jnp.full((16,), j, jnp.int32)])
            for k in range(HH // 16):
                rows_v[j, pl.ds(k * 16, 16)] = rows_v[j, pl.ds(k * 16, 16)] * bb
            return 0
        lax.fori_loop(0, C, mul_body, 0)
        pltpu.sync_copy(rows_v, acc_sh.at[didx_v], add=True)
        return 0
    lax.fori_loop(0, NCH, chunk_body, 0)
    plsc.subcore_barrier()

    @pl.when(ss < NS - 1)
    def _():
        pltpu.sync_copy(acc_sh.at[pl.ds(ss * RB, RB)],
                        out_hbm.at[cc].at[pl.ds(ss * RB, RB)])

    @pl.when(ss == NS - 1)
    def _():
        pltpu.sync_copy(acc_sh.at[pl.ds(ss * RB, 640)],
                        out_hbm.at[cc].at[pl.ds(ss * RB, 640)])


def _layer_body(acc_ref, hw_ref, dinv_ref, dinv2_ref, b_ref, g_ref, bb_ref,
                wn_ref, out_ref):
    a = jnp.concatenate([acc_ref[0], acc_ref[1]], axis=1)
    hw = jnp.concatenate([hw_ref[0], hw_ref[1]], axis=1)
    h = dinv_ref[...] * a + dinv2_ref[...] * hw + b_ref[...]
    m = jnp.mean(h, axis=0, keepdims=True)
    v = jnp.mean((h - m) ** 2, axis=0, keepdims=True)
    h = (h - m) * lax.rsqrt(v + 1e-5) * g_ref[...] + bb_ref[...]
    h = jnp.where(h >= 0, h, jnp.float32(ALPHA) * h)
    o = jnp.dot(h, wn_ref[...], preferred_element_type=jnp.float32)
    out_ref[0] = o[:, :HH]
    out_ref[1] = o[:, HH:]


_layer_call = pl.pallas_call(
    _layer_body,
    out_shape=jax.ShapeDtypeStruct((2, N, HH), jnp.float32),
)


def _final_body(acc_ref, hw_ref, dinv_ref, dinv2_ref, b_ref, g_ref, bb_ref,
                bidx_ref, w1_ref, b1_ref, g1_ref, bb1_ref,
                w2_ref, b2_ref, g2_ref, bb2_ref, w3_ref, b3_ref, out_ref):
    a = jnp.concatenate([acc_ref[0], acc_ref[1]], axis=1)
    hw = jnp.concatenate([hw_ref[0], hw_ref[1]], axis=1)
    h = dinv_ref[...] * a + dinv2_ref[...] * hw + b_ref[...]
    m = jnp.mean(h, axis=0, keepdims=True)
    v = jnp.mean((h - m) ** 2, axis=0, keepdims=True)
    h = (h - m) * lax.rsqrt(v + 1e-5) * g_ref[...] + bb_ref[...]
    h = jnp.where(h >= 0, h, jnp.float32(ALPHA) * h)
    cols = lax.broadcasted_iota(jnp.int32, (N, G), 1)
    bo = (bidx_ref[...] == cols).astype(jnp.float32)
    gp = lax.dot_general(bo, h, (((0,), (0,)), ((), ())),
                         preferred_element_type=jnp.float32)
    z = jnp.dot(gp, w1_ref[...], preferred_element_type=jnp.float32) + b1_ref[...]
    m = jnp.mean(z, axis=0, keepdims=True)
    v = jnp.mean((z - m) ** 2, axis=0, keepdims=True)
    z = (z - m) * lax.rsqrt(v + 1e-5) * g1_ref[...] + bb1_ref[...]
    z = jnp.maximum(z, 0.0)
    z = jnp.dot(z, w2_ref[...], preferred_element_type=jnp.float32) + b2_ref[...]
    m = jnp.mean(z, axis=0, keepdims=True)
    v = jnp.mean((z - m) ** 2, axis=0, keepdims=True)
    z = (z - m) * lax.rsqrt(v + 1e-5) * g2_ref[...] + bb2_ref[...]
    z = jnp.maximum(z, 0.0)
    out_ref[...] = jnp.dot(z, w3_ref[...], preferred_element_type=jnp.float32) + b3_ref[...]


_final_call = pl.pallas_call(
    _final_body,
    out_shape=jax.ShapeDtypeStruct((G, G), jnp.float32),
)


def kernel(params, x, edge_index, edge_attr, batch_idx):
    src = edge_index[0].astype(jnp.int32)
    dst = edge_index[1].astype(jnp.int32)
    ea_t = edge_attr.astype(jnp.int32).T
    x32 = x.astype(jnp.int32)
    we = params["W_edge"]
    be = params["b_edge"].reshape(1, 1)

    hw0, ew_row = _pre_call(x32, ea_t, params["gc_W"][0], we, be)
    ew = ew_row.reshape(E)

    dinv, dinv2, ews = _sc_prep(src, dst, ew)
    dinv_c = dinv.reshape(N, 1)
    dinv2_c = dinv2.reshape(N, 1)

    hw = hw0
    for i in range(3):
        acc = _sc_agg(hw, src, dst, ews)
        hw = _layer_call(acc, hw,
                         dinv_c, dinv2_c,
                         params["gc_b"][i].reshape(1, H),
                         params["bn_g"][i].reshape(1, H),
                         params["bn_b"][i].reshape(1, H),
                         params["gc_W"][i + 1])
    acc = _sc_agg(hw, src, dst, ews)

    h2 = H // 2
    w3p = jnp.zeros((h2, G), jnp.float32).at[:, :2].set(params["mlp_W"][2])
    b3p = jnp.zeros((1, G), jnp.float32).at[0, :2].set(params["mlp_b"][2])
    out = _final_call(acc, hw, dinv_c, dinv2_c,
                      params["gc_b"][3].reshape(1, H),
                      params["bn_g"][3].reshape(1, H),
                      params["bn_b"][3].reshape(1, H),
                      batch_idx.astype(jnp.int32).reshape(N, 1),
                      params["mlp_W"][0], params["mlp_b"][0].reshape(1, h2),
                      params["mlp_bn_g"][0].reshape(1, h2),
                      params["mlp_bn_b"][0].reshape(1, h2),
                      params["mlp_W"][1], params["mlp_b"][1].reshape(1, h2),
                      params["mlp_bn_g"][1].reshape(1, h2),
                      params["mlp_bn_b"][1].reshape(1, h2),
                      w3p, b3p)
    return out[:, :2]
